# baseline (speedup 1.0000x reference)
import functools

import jax
import jax.numpy as jnp
from jax import lax
from jax.experimental import pallas as pl
from jax.experimental.pallas import tpu as pltpu
from jax.experimental.pallas import tpu_sc as plsc

VOCAB = 1000000
D = 64
BATCH = 4096
HIST = 200
B_TOTAL = BATCH * HIST

NC, NS = 2, 16
NW = NC * NS
B_PER_W = B_TOTAL // NW
SUB = 128
CHUNK = 1024
N_SUB = CHUNK // SUB
N_CHUNK = B_PER_W // CHUNK


def _build():
  mesh = plsc.VectorSubcoreMesh(
      core_axis_name="c", subcore_axis_name="s", num_cores=NC, num_subcores=NS)

  @functools.partial(
      pl.kernel,
      mesh=mesh,
      out_type=jax.ShapeDtypeStruct((B_TOTAL, D), jnp.float32),
      scratch_types=[
          pltpu.VMEM((N_SUB, SUB), jnp.int32),
          pltpu.VMEM((CHUNK, D), jnp.float32),
          pltpu.SemaphoreType.DMA,
      ],
      compiler_params=pltpu.CompilerParams(use_tc_tiling_on_sc=False),
  )
  def emb_kernel(idx_hbm, table_hbm, out_hbm, idx_v, rows_v, sem):
    wid = lax.axis_index("s") * NC + lax.axis_index("c")
    row0 = wid * B_PER_W

    def body(g, carry):
      base = pl.multiple_of(row0 + g * CHUNK, CHUNK)
      pltpu.sync_copy(idx_hbm.at[pl.ds(pl.multiple_of(base // SUB, N_SUB), N_SUB)], idx_v)
      handles = []
      for j in range(N_SUB):
        handles.append(pltpu.async_copy(
            table_hbm.at[idx_v.at[j]],
            rows_v.at[pl.ds(j * SUB, SUB)],
            sem))
      for h in handles:
        h.wait()
      pltpu.sync_copy(rows_v, out_hbm.at[pl.ds(base, CHUNK)])
      return carry

    lax.fori_loop(0, N_CHUNK, body, 0)

  return emb_kernel


_emb = _build()


def kernel(token_ids, table):
  idx = token_ids.reshape(B_TOTAL // SUB, SUB).astype(jnp.int32)
  out = _emb(idx, table)
  return out.reshape(BATCH, HIST, D)


# speedup vs baseline: 1.0093x; 1.0093x over previous
import functools

import jax
import jax.numpy as jnp
from jax import lax
from jax.experimental import pallas as pl
from jax.experimental.pallas import tpu as pltpu
from jax.experimental.pallas import tpu_sc as plsc

VOCAB = 1000000
D = 64
BATCH = 4096
HIST = 200
B_TOTAL = BATCH * HIST

NC, NS = 2, 16
NW = NC * NS
B_PER_W = B_TOTAL // NW
SUB = 128
IDX_ROWS = B_PER_W // SUB
CHUNK = 512
N_SUB = CHUNK // SUB
N_CHUNK = B_PER_W // CHUNK
N2 = N_CHUNK // 2


def _build():
  mesh = plsc.VectorSubcoreMesh(
      core_axis_name="c", subcore_axis_name="s", num_cores=NC, num_subcores=NS)

  @functools.partial(
      pl.kernel,
      mesh=mesh,
      out_type=jax.ShapeDtypeStruct((B_TOTAL, D), jnp.float32),
      scratch_types=[
          pltpu.VMEM((IDX_ROWS, SUB), jnp.int32),
          pltpu.VMEM((CHUNK, D), jnp.float32),
          pltpu.VMEM((CHUNK, D), jnp.float32),
          pltpu.SemaphoreType.DMA,
          pltpu.SemaphoreType.DMA,
          pltpu.SemaphoreType.DMA,
          pltpu.SemaphoreType.DMA,
      ],
      compiler_params=pltpu.CompilerParams(use_tc_tiling_on_sc=False),
  )
  def emb_kernel(idx_hbm, table_hbm, out_hbm, idx_v, buf0, buf1, g0, g1, o0, o1):
    wid = lax.axis_index("s") * NC + lax.axis_index("c")
    row0 = pl.multiple_of(wid * B_PER_W, B_PER_W)

    pltpu.sync_copy(idx_hbm.at[pl.ds(pl.multiple_of(wid * IDX_ROWS, 8), IDX_ROWS)],
                    idx_v)

    def fire_gather(g, buf, sem):
      hs = []
      for j in range(N_SUB):
        hs.append(pltpu.async_copy(
            table_hbm.at[idx_v.at[g * N_SUB + j]],
            buf.at[pl.ds(j * SUB, SUB)],
            sem))
      return hs

    def fire_wb(g, buf, sem):
      return pltpu.async_copy(
          buf, out_hbm.at[pl.ds(pl.multiple_of(row0 + g * CHUNK, CHUNK), CHUNK)],
          sem)

    fire_gather(0, buf0, g0)
    fire_gather(1, buf1, g1)

    def body(k, carry):
      ge = k * 2
      for j in range(N_SUB):
        pltpu.make_async_copy(
            table_hbm.at[idx_v.at[ge * N_SUB + j]],
            buf0.at[pl.ds(j * SUB, SUB)], g0).wait()
      wb0 = fire_wb(ge, buf0, o0)
      for j in range(N_SUB):
        pltpu.make_async_copy(
            table_hbm.at[idx_v.at[(ge + 1) * N_SUB + j]],
            buf1.at[pl.ds(j * SUB, SUB)], g1).wait()
      wb1 = fire_wb(ge + 1, buf1, o1)
      wb0.wait()
      fire_gather(ge + 2, buf0, g0)
      wb1.wait()
      fire_gather(ge + 3, buf1, g1)
      return carry

    lax.fori_loop(0, N2 - 1, body, 0)

    ge = (N2 - 1) * 2
    for j in range(N_SUB):
      pltpu.make_async_copy(
          table_hbm.at[idx_v.at[ge * N_SUB + j]],
          buf0.at[pl.ds(j * SUB, SUB)], g0).wait()
    wb0 = fire_wb(ge, buf0, o0)
    for j in range(N_SUB):
      pltpu.make_async_copy(
          table_hbm.at[idx_v.at[(ge + 1) * N_SUB + j]],
          buf1.at[pl.ds(j * SUB, SUB)], g1).wait()
    wb1 = fire_wb(ge + 1, buf1, o1)
    wb0.wait()
    wb1.wait()

  return emb_kernel


_emb = _build()


def kernel(token_ids, table):
  idx = token_ids.reshape(B_TOTAL // SUB, SUB).astype(jnp.int32)
  out = _emb(idx, table)
  return out.reshape(BATCH, HIST, D)
